# baseline (device time: 151898 ns/iter reference)
import jax
import jax.numpy as jnp
from jax import lax
from jax.experimental import pallas as pl
from jax.experimental.pallas import tpu as pltpu

NT = 8


def kernel(x, W):
    t, d = x.shape
    _, v_loc = W.shape
    v_glob = 2 * v_loc
    tile = v_loc // NT
    half = t // 2

    def body(
        x_ref, w_ref, out_ref, logits, sbuf, xrbuf, yrbuf,
        w_tiles, wsems, csems, cpsems, sems,
    ):
        mx = lax.axis_index("x")
        my = lax.axis_index("y")
        nbr = (1 - mx, my)
        nbr_y = (mx, 1 - my)
        r0 = my * half
        q0 = (1 - my) * half
        own_c0 = mx * v_loc
        oth_c0 = (1 - mx) * v_loc

        barrier = pltpu.get_barrier_semaphore()
        for nb in (nbr, nbr_y):
            pl.semaphore_signal(
                barrier, inc=1, device_id=nb,
                device_id_type=pl.DeviceIdType.MESH,
            )
        pl.semaphore_wait(barrier, 2)

        xv = x_ref[...]

        def wdma(j, slot):
            return pltpu.make_async_copy(
                w_ref.at[:, pl.ds(j * tile, tile)],
                w_tiles.at[slot],
                wsems.at[slot],
            )

        def stage_cp(j):
            cs = slice(j * tile, (j + 1) * tile)
            return pltpu.make_async_copy(
                logits.at[pl.ds(r0, half), cs], sbuf.at[:, cs], csems.at[j]
            )

        def x_rdma(j):
            cs = slice(j * tile, (j + 1) * tile)
            return pltpu.make_async_remote_copy(
                src_ref=sbuf.at[:, cs],
                dst_ref=xrbuf.at[:, cs],
                send_sem=sems.at[0, j],
                recv_sem=sems.at[1, j],
                device_id=nbr,
                device_id_type=pl.DeviceIdType.MESH,
            )

        def fwd_rdma(j):
            cs = slice(j * tile, (j + 1) * tile)
            return pltpu.make_async_remote_copy(
                src_ref=xrbuf.at[:, cs],
                dst_ref=yrbuf.at[:, cs],
                send_sem=sems.at[2, j],
                recv_sem=sems.at[3, j],
                device_id=nbr_y,
                device_id_type=pl.DeviceIdType.MESH,
            )

        wdma(0, 0).start()
        m_loc = jnp.full((t, 1), -1e30, jnp.float32)
        for j in range(NT):
            slot = j % 2
            if j + 1 < NT:
                wdma(j + 1, 1 - slot).start()
            wdma(j, slot).wait()
            cs = slice(j * tile, (j + 1) * tile)
            tl = jnp.dot(xv, w_tiles[slot], preferred_element_type=jnp.float32)
            logits[:, cs] = tl
            m_loc = jnp.maximum(m_loc, jnp.max(tl, axis=1, keepdims=True))
            stage_cp(j).start()
            if j > 0:
                stage_cp(j - 1).wait()
                x_rdma(j - 1).start()
        stage_cp(NT - 1).wait()
        x_rdma(NT - 1).start()

        s_loc = jnp.zeros((t, 1), jnp.float32)
        for j in range(NT):
            cs = slice(j * tile, (j + 1) * tile)
            x_rdma(j).wait_recv()
            fwd_rdma(j).start()
            e_t = jnp.exp(logits[:, cs] - m_loc)
            logits[:, cs] = e_t
            s_loc = s_loc + jnp.sum(e_t, axis=1, keepdims=True)

        m_xq = jnp.full((half, 1), -1e30, jnp.float32)
        for j in range(NT):
            cs = slice(j * tile, (j + 1) * tile)
            m_xq = jnp.maximum(
                m_xq, jnp.max(xrbuf[:, cs], axis=1, keepdims=True)
            )
        s_xq = jnp.zeros((half, 1), jnp.float32)
        for j in range(NT):
            cs = slice(j * tile, (j + 1) * tile)
            s_xq = s_xq + jnp.sum(
                jnp.exp(xrbuf[:, cs] - m_xq), axis=1, keepdims=True
            )

        for j in range(NT):
            fwd_rdma(j).wait_recv()
        m_yq = jnp.full((half, 1), -1e30, jnp.float32)
        for j in range(NT):
            cs = slice(j * tile, (j + 1) * tile)
            m_yq = jnp.maximum(
                m_yq, jnp.max(yrbuf[:, cs], axis=1, keepdims=True)
            )
        s_yq = jnp.zeros((half, 1), jnp.float32)
        for j in range(NT):
            cs = slice(j * tile, (j + 1) * tile)
            s_yq = s_yq + jnp.sum(
                jnp.exp(yrbuf[:, cs] - m_yq), axis=1, keepdims=True
            )

        is_y0 = my == 0
        m_rem = jnp.where(
            is_y0,
            jnp.concatenate([m_xq, m_yq], axis=0),
            jnp.concatenate([m_yq, m_xq], axis=0),
        )
        s_rem = jnp.where(
            is_y0,
            jnp.concatenate([s_xq, s_yq], axis=0),
            jnp.concatenate([s_yq, s_xq], axis=0),
        )
        m_glob = jnp.maximum(m_loc, m_rem)
        s_glob = s_loc * jnp.exp(m_loc - m_glob) + s_rem * jnp.exp(
            m_rem - m_glob
        )
        inv_s = 1.0 / s_glob
        scale_own = jnp.exp(m_loc - m_glob) * inv_s
        mg_x = jnp.where(is_y0, m_glob[0:half, :], m_glob[half:t, :])
        iv_x = jnp.where(is_y0, inv_s[0:half, :], inv_s[half:t, :])
        mg_y = jnp.where(is_y0, m_glob[half:t, :], m_glob[0:half, :])
        iv_y = jnp.where(is_y0, inv_s[half:t, :], inv_s[0:half, :])

        for j in range(NT):
            cs = slice(j * tile, (j + 1) * tile)
            logits[:, cs] = logits[:, cs] * scale_own
        cp_own = pltpu.make_async_copy(
            logits, out_ref.at[:, pl.ds(own_c0, v_loc)], cpsems.at[0]
        )
        cp_own.start()

        for j in range(NT):
            fwd_rdma(j).wait_send()
        for j in range(NT):
            cs = slice(j * tile, (j + 1) * tile)
            xrbuf[:, cs] = jnp.exp(xrbuf[:, cs] - mg_x) * iv_x
        cp_x = pltpu.make_async_copy(
            xrbuf,
            out_ref.at[pl.ds(r0, half), pl.ds(oth_c0, v_loc)],
            cpsems.at[1],
        )
        cp_x.start()

        for j in range(NT):
            cs = slice(j * tile, (j + 1) * tile)
            yrbuf[:, cs] = jnp.exp(yrbuf[:, cs] - mg_y) * iv_y
        cp_y = pltpu.make_async_copy(
            yrbuf,
            out_ref.at[pl.ds(q0, half), pl.ds(oth_c0, v_loc)],
            cpsems.at[2],
        )
        cp_y.start()

        for j in range(NT):
            x_rdma(j).wait_send()
        cp_own.wait()
        cp_x.wait()
        cp_y.wait()

    return pl.pallas_call(
        body,
        out_shape=jax.ShapeDtypeStruct((t, v_glob), jnp.float32),
        in_specs=[
            pl.BlockSpec(memory_space=pltpu.VMEM),
            pl.BlockSpec(memory_space=pl.ANY),
        ],
        out_specs=pl.BlockSpec(memory_space=pl.ANY),
        scratch_shapes=[
            pltpu.VMEM((t, v_loc), jnp.float32),
            pltpu.VMEM((half, v_loc), jnp.float32),
            pltpu.VMEM((half, v_loc), jnp.float32),
            pltpu.VMEM((half, v_loc), jnp.float32),
            pltpu.VMEM((2, d, tile), jnp.float32),
            pltpu.SemaphoreType.DMA((2,)),
            pltpu.SemaphoreType.DMA((NT,)),
            pltpu.SemaphoreType.DMA((3,)),
            pltpu.SemaphoreType.DMA((4, NT)),
        ],
        compiler_params=pltpu.CompilerParams(
            collective_id=0,
            vmem_limit_bytes=62 * 1024 * 1024,
        ),
    )(x, W)


# device time: 150668 ns/iter; 1.0082x vs baseline; 1.0082x over previous
import jax
import jax.numpy as jnp
from jax import lax
from jax.experimental import pallas as pl
from jax.experimental.pallas import tpu as pltpu

N_TILES = 8
NC = 16


def kernel(x, W):
    t, d = x.shape
    _, v_loc = W.shape
    v_glob = 2 * v_loc
    tile = v_loc // N_TILES

    def body(
        x_ref, w_ref, out_ref, w_tiles, logits, sbuf, xrbuf, stat_tx,
        stat_rx, wsems, csem, cpsems, stat_sems, sems,
    ):
        mx = lax.axis_index("x")
        my = lax.axis_index("y")
        nbr = (1 - mx, my)
        nbr_y = (mx, 1 - my)
        half_rows = t // 2
        rc = half_rows // NC
        r0 = my * half_rows
        q0 = (1 - my) * half_rows
        own_c0 = mx * v_loc
        oth_c0 = (1 - mx) * v_loc

        barrier = pltpu.get_barrier_semaphore()
        for nb in (nbr, nbr_y):
            pl.semaphore_signal(
                barrier, inc=1, device_id=nb,
                device_id_type=pl.DeviceIdType.MESH,
            )
        pl.semaphore_wait(barrier, 2)

        xv = x_ref[...]

        def wdma(i, slot):
            return pltpu.make_async_copy(
                w_ref.at[:, pl.ds(i * tile, tile)],
                w_tiles.at[slot],
                wsems.at[slot],
            )

        wdma(0, 0).start()
        m_loc = jnp.full((t, 1), -1e30, jnp.float32)
        for i in range(N_TILES):
            slot = i % 2
            if i + 1 < N_TILES:
                wdma(i + 1, 1 - slot).start()
            wdma(i, slot).wait()
            tl = jnp.dot(xv, w_tiles[slot], preferred_element_type=jnp.float32)
            logits[:, i * tile : (i + 1) * tile] = tl
            m_loc = jnp.maximum(m_loc, jnp.max(tl, axis=1, keepdims=True))

        s_loc = jnp.zeros((t, 1), jnp.float32)
        for i in range(N_TILES):
            sl = slice(i * tile, (i + 1) * tile)
            e_t = jnp.exp(logits[:, sl] - m_loc)
            logits[:, sl] = e_t
            s_loc = s_loc + jnp.sum(e_t, axis=1, keepdims=True)

        stat_tx[:, 0:128] = jnp.broadcast_to(m_loc, (t, 128))
        stat_tx[:, 128:256] = jnp.broadcast_to(s_loc, (t, 128))
        stat_rdma = pltpu.make_async_remote_copy(
            src_ref=stat_tx,
            dst_ref=stat_rx,
            send_sem=stat_sems.at[0],
            recv_sem=stat_sems.at[1],
            device_id=nbr,
            device_id_type=pl.DeviceIdType.MESH,
        )
        stat_rdma.start()

        stage = pltpu.make_async_copy(
            logits.at[pl.ds(r0, half_rows), :], sbuf, csem
        )
        stage.start()
        stage.wait()
        x_out = []
        for c in range(NC):
            cs = slice(c * rc, (c + 1) * rc)
            rd = pltpu.make_async_remote_copy(
                src_ref=sbuf.at[cs, :],
                dst_ref=xrbuf.at[cs, :],
                send_sem=sems.at[0, c],
                recv_sem=sems.at[1, c],
                device_id=nbr,
                device_id_type=pl.DeviceIdType.MESH,
            )
            rd.start()
            x_out.append(rd)

        stat_rdma.wait()

        m_rem = stat_rx[:, 0:1]
        s_rem = stat_rx[:, 128:129]
        m_glob = jnp.maximum(m_loc, m_rem)
        s_glob = s_loc * jnp.exp(m_loc - m_glob) + s_rem * jnp.exp(
            m_rem - m_glob
        )
        inv_s = 1.0 / s_glob
        scale = jnp.exp(m_loc - m_glob) * inv_s

        for i in range(N_TILES):
            sl = slice(i * tile, (i + 1) * tile)
            logits[:, sl] = logits[:, sl] * scale

        local_cp = pltpu.make_async_copy(
            logits, out_ref.at[:, pl.ds(own_c0, v_loc)], cpsems.at[0]
        )
        local_cp.start()

        corr = jnp.exp(m_rem - m_glob) * inv_s
        is_y0 = my == 0
        corr_x = jnp.where(is_y0, corr[0:half_rows, :], corr[half_rows:t, :])

        y_out = []
        for c in range(NC):
            cs = slice(c * rc, (c + 1) * rc)
            x_in = pltpu.make_async_remote_copy(
                src_ref=sbuf.at[cs, :],
                dst_ref=xrbuf.at[cs, :],
                send_sem=sems.at[0, c],
                recv_sem=sems.at[1, c],
                device_id=nbr,
                device_id_type=pl.DeviceIdType.MESH,
            )
            x_in.wait_recv()
            xrbuf[cs, :] = xrbuf[cs, :] * corr_x[cs, :]
            fwd = pltpu.make_async_remote_copy(
                src_ref=xrbuf.at[cs, :],
                dst_ref=out_ref.at[
                    pl.ds(r0 + c * rc, rc), pl.ds(oth_c0, v_loc)
                ],
                send_sem=sems.at[2, c],
                recv_sem=sems.at[3, c],
                device_id=nbr_y,
                device_id_type=pl.DeviceIdType.MESH,
            )
            fwd.start()
            y_out.append(fwd)

        cp_x = pltpu.make_async_copy(
            xrbuf,
            out_ref.at[pl.ds(r0, half_rows), pl.ds(oth_c0, v_loc)],
            cpsems.at[1],
        )
        cp_x.start()

        for c in range(NC):
            y_in = pltpu.make_async_remote_copy(
                src_ref=xrbuf.at[slice(c * rc, (c + 1) * rc), :],
                dst_ref=out_ref.at[
                    pl.ds(q0 + c * rc, rc), pl.ds(oth_c0, v_loc)
                ],
                send_sem=sems.at[2, c],
                recv_sem=sems.at[3, c],
                device_id=nbr_y,
                device_id_type=pl.DeviceIdType.MESH,
            )
            y_in.wait_recv()
        for rd in x_out:
            rd.wait_send()
        for rd in y_out:
            rd.wait_send()
        local_cp.wait()
        cp_x.wait()

    return pl.pallas_call(
        body,
        out_shape=jax.ShapeDtypeStruct((t, v_glob), jnp.float32),
        in_specs=[
            pl.BlockSpec(memory_space=pltpu.VMEM),
            pl.BlockSpec(memory_space=pl.ANY),
        ],
        out_specs=pl.BlockSpec(memory_space=pl.ANY),
        scratch_shapes=[
            pltpu.VMEM((2, d, tile), jnp.float32),
            pltpu.VMEM((t, v_loc), jnp.float32),
            pltpu.VMEM((t // 2, v_loc), jnp.float32),
            pltpu.VMEM((t // 2, v_loc), jnp.float32),
            pltpu.VMEM((t, 256), jnp.float32),
            pltpu.VMEM((t, 256), jnp.float32),
            pltpu.SemaphoreType.DMA((2,)),
            pltpu.SemaphoreType.DMA,
            pltpu.SemaphoreType.DMA((2,)),
            pltpu.SemaphoreType.DMA((2,)),
            pltpu.SemaphoreType.DMA((4, NC)),
        ],
        compiler_params=pltpu.CompilerParams(
            collective_id=0,
            vmem_limit_bytes=62 * 1024 * 1024,
        ),
    )(x, W)
